# Initial kernel scaffold; baseline (speedup 1.0000x reference)
import jax
import jax.numpy as jnp
from jax.experimental import pallas as pl


def kernel(xyz, points, W0, b0, g0, beta0, W1, b1, g1, beta1, W2, b2, g2, beta2):
    raise NotImplementedError("write your pallas kernel here")


# speedup vs baseline: 10.0400x; 10.0400x over previous
import functools

import jax
import jax.numpy as jnp
from jax import lax
from jax.experimental import pallas as pl
from jax.experimental.pallas import tpu as pltpu
from jax.experimental.pallas import tpu_sc as plsc

_B, _N, _S, _K = 8, 4096, 512, 32
_R2 = 0.2 * 0.2
_CH = 64
_NC, _NS = 2, 16
_NW = _NC * _NS
_CPW = (_B * _S) // _NW
_ROWS = _B * _S * _K
_RPW = _CPW * _K
_NCH = _N // 16


def _fps_body(xt_ref, f0_ref, nxt_ref, dist_ref):
    X = xt_ref[0]
    Y = xt_ref[1]
    Z = xt_ref[2]
    lane = lax.broadcasted_iota(jnp.int32, (_B, _N), 1)
    sidx = lax.broadcasted_iota(jnp.int32, (_B, _S), 1)
    dist_ref[...] = jnp.full((_B, _N), 1e10, jnp.float32)
    nxt_ref[...] = jnp.zeros((3, _B, _S), jnp.float32)

    def step(i, far):
        oh = lane == far
        cx = jnp.sum(jnp.where(oh, X, 0.0), axis=1, keepdims=True)
        cy = jnp.sum(jnp.where(oh, Y, 0.0), axis=1, keepdims=True)
        cz = jnp.sum(jnp.where(oh, Z, 0.0), axis=1, keepdims=True)
        rec = sidx == i
        nxt_ref[0] = jnp.where(rec, cx, nxt_ref[0])
        nxt_ref[1] = jnp.where(rec, cy, nxt_ref[1])
        nxt_ref[2] = jnp.where(rec, cz, nxt_ref[2])
        dx = X - cx
        dy = Y - cy
        dz = Z - cz
        d = dx * dx + dy * dy + dz * dz
        dm = jnp.minimum(dist_ref[...], d)
        dist_ref[...] = dm
        m = jnp.max(dm, axis=1, keepdims=True)
        return jnp.min(jnp.where(dm == m, lane, _N), axis=1, keepdims=True)

    lax.fori_loop(0, _S, step, f0_ref[...])


def _fps(xt, far0):
    return pl.pallas_call(
        _fps_body,
        out_shape=jax.ShapeDtypeStruct((3, _B, _S), jnp.float32),
        scratch_shapes=[pltpu.VMEM((_B, _N), jnp.float32)],
    )(xt, far0)


def _bf16r(x):
    b = plsc.bitcast(x, jnp.int32)
    r = (b + 0x7FFF + ((b >> 16) & 1)) & jnp.int32(-65536)
    return plsc.bitcast(r, jnp.float32)


def _scq_body(xt_hbm, nxt_hbm, pts_hbm, xg_hbm, xp_hbm,
              xs_v, ys_v, zs_v, x2_v, xb_v, yb_v, zb_v,
              cx_v, cy_v, cz_v,
              idxbuf_v, idx_all_v, xg_v, pbuf_v, sem):
    w = lax.axis_index("s") * _NC + lax.axis_index("c")
    b = w // (_NW // _B)
    s0 = (w % (_NW // _B)) * _CPW
    boff = b * _N

    pltpu.sync_copy(xt_hbm.at[0, b], xs_v)
    pltpu.sync_copy(xt_hbm.at[1, b], ys_v)
    pltpu.sync_copy(xt_hbm.at[2, b], zs_v)
    pltpu.sync_copy(nxt_hbm.at[0, b, pl.ds(s0, _CPW)], cx_v)
    pltpu.sync_copy(nxt_hbm.at[1, b, pl.ds(s0, _CPW)], cy_v)
    pltpu.sync_copy(nxt_hbm.at[2, b, pl.ds(s0, _CPW)], cz_v)

    zeros16 = jnp.zeros((16,), jnp.float32)

    def x2_step(t, _):
        sl = pl.ds(t * 16, 16)
        xs = xs_v[sl]
        ys = ys_v[sl]
        zs = zs_v[sl]
        x2_v[sl] = (xs * xs + ys * ys) + zs * zs
        xb_v[sl] = _bf16r(xs)
        yb_v[sl] = _bf16r(ys)
        zb_v[sl] = _bf16r(zs)
        return 0

    lax.fori_loop(0, _NCH, x2_step, 0)

    def zz_step(t, _):
        xg_v[t] = zeros16
        return 0

    lax.fori_loop(0, _RPW, zz_step, 0)

    iota16 = lax.iota(jnp.int32, 16)

    def sel_step(cent, _):
        csplat = jnp.full((16,), cent, jnp.int32)
        cxs = plsc.load_gather(cx_v, [csplat])
        cys = plsc.load_gather(cy_v, [csplat])
        czs = plsc.load_gather(cz_v, [csplat])
        c2s = (cxs * cxs + cys * cys) + czs * czs
        cxb = _bf16r(cxs)
        cyb = _bf16r(cys)
        czb = _bf16r(czs)

        def cond(st):
            j, cnt = st
            return jnp.logical_and(cnt < _K, j < _NCH)

        def body(st):
            j, cnt = st
            sl = pl.ds(j * 16, 16)
            xb = xb_v[sl]
            yb = yb_v[sl]
            zb = zb_v[sl]
            x2 = x2_v[sl]
            dot = (cxb * xb + cyb * yb) + czb * zb
            sq = (c2s + x2) - 2.0 * dot
            mask = jnp.logical_not(sq > _R2)
            plsc.store_compressed(idxbuf_v.at[pl.ds(cnt, 16)],
                                  iota16 + j * 16, mask=mask)
            cnt = cnt + jnp.sum(jnp.where(mask, 1, 0))
            return j + 1, cnt

        _, cnt = lax.while_loop(cond, body, (0, 0))
        first = plsc.load_gather(idxbuf_v, [csplat * 0])
        row = cent // 4
        col = (cent % 4) * _K
        for h in range(2):
            v = idxbuf_v[pl.ds(h * 16, 16)]
            valid = (iota16 + h * 16) < cnt
            v = jnp.where(valid, v, first)
            gx = plsc.load_gather(xs_v, [v]) - cxs
            gy = plsc.load_gather(ys_v, [v]) - cys
            gz = plsc.load_gather(zs_v, [v]) - czs
            rows = iota16 + (cent * _K + h * 16)
            plsc.store_scatter(xg_v, [rows, jnp.zeros((16,), jnp.int32)], gx)
            plsc.store_scatter(xg_v, [rows, jnp.full((16,), 1, jnp.int32)], gy)
            plsc.store_scatter(xg_v, [rows, jnp.full((16,), 2, jnp.int32)], gz)
            idx_all_v[row, pl.ds(col + h * 16, 16)] = v + boff
        return 0

    lax.fori_loop(0, _CPW, sel_step, 0)

    def gat_step(j, _):
        pltpu.async_copy(pts_hbm.at[idx_all_v.at[j]], pbuf_v, sem).wait()
        pltpu.sync_copy(pbuf_v, xp_hbm.at[pl.ds(w * _RPW + j * 128, 128)])
        return 0

    lax.fori_loop(0, _RPW // 128, gat_step, 0)
    pltpu.sync_copy(xg_v, xg_hbm.at[pl.ds(w * _RPW, _RPW)])


_scq_cache = []


def _scq(xt, nxt, pts):
    if not _scq_cache:
        _scq_cache.append(_build_scq())
    return _scq_cache[0](xt, nxt, pts)


def _build_scq():
    return functools.partial(
        pl.kernel,
        out_type=[jax.ShapeDtypeStruct((_ROWS, 16), jnp.float32),
                  jax.ShapeDtypeStruct((_ROWS, _CH), jnp.float32)],
        mesh=plsc.VectorSubcoreMesh(core_axis_name="c", subcore_axis_name="s",
                                    num_cores=_NC, num_subcores=_NS),
        compiler_params=pltpu.CompilerParams(needs_layout_passes=False,
                                             use_tc_tiling_on_sc=False),
        scratch_types=[
        pltpu.VMEM((_N,), jnp.float32),
        pltpu.VMEM((_N,), jnp.float32),
        pltpu.VMEM((_N,), jnp.float32),
        pltpu.VMEM((_N,), jnp.float32),
        pltpu.VMEM((_N,), jnp.float32),
        pltpu.VMEM((_N,), jnp.float32),
        pltpu.VMEM((_N,), jnp.float32),
        pltpu.VMEM((_CPW,), jnp.float32),
        pltpu.VMEM((_CPW,), jnp.float32),
        pltpu.VMEM((_CPW,), jnp.float32),
        pltpu.VMEM((48,), jnp.int32),
        pltpu.VMEM((_RPW // 128, 128), jnp.int32),
        pltpu.VMEM((_RPW, 16), jnp.float32),
            pltpu.VMEM((128, _CH), jnp.float32),
            pltpu.SemaphoreType.DMA,
        ],
    )(_scq_body)


_RT = 1024
_GRID = _ROWS // _RT


def _stats_update(acc_ref, z):
    part = jnp.concatenate([jnp.sum(z, axis=0, keepdims=True),
                            jnp.sum(z * z, axis=0, keepdims=True)], axis=0)

    @pl.when(pl.program_id(0) == 0)
    def _():
        acc_ref[...] = jnp.zeros_like(acc_ref)

    acc_ref[...] += part


def _scale_shift(acc_ref, gb_ref):
    mean = acc_ref[0:1, :] * (1.0 / _ROWS)
    var = acc_ref[1:2, :] * (1.0 / _ROWS) - mean * mean
    scale = gb_ref[0:1, :] * lax.rsqrt(var + 1e-5)
    shift = gb_ref[1:2, :] - mean * scale
    return scale, shift


def _m1_body(xg_ref, xp_ref, wg_ref, wp_ref, z_ref, acc_ref):
    z = jnp.dot(xg_ref[...], wg_ref[...], preferred_element_type=jnp.float32)
    z = z + jnp.dot(xp_ref[...], wp_ref[...], preferred_element_type=jnp.float32)
    z_ref[...] = z
    _stats_update(acc_ref, z)


def _m1(xg, xp, wg, wp):
    return pl.pallas_call(
        _m1_body,
        grid=(_GRID,),
        in_specs=[
            pl.BlockSpec((_RT, 16), lambda i: (i, 0)),
            pl.BlockSpec((_RT, _CH), lambda i: (i, 0)),
            pl.BlockSpec((16, 128), lambda i: (0, 0)),
            pl.BlockSpec((_CH, 128), lambda i: (0, 0)),
        ],
        out_specs=[
            pl.BlockSpec((_RT, 128), lambda i: (i, 0)),
            pl.BlockSpec((2, 128), lambda i: (0, 0)),
        ],
        out_shape=[jax.ShapeDtypeStruct((_ROWS, 128), jnp.float32),
                   jax.ShapeDtypeStruct((2, 128), jnp.float32)],
    )(xg, xp, wg, wp)


def _mm_body(z_ref, acc_ref, gb_ref, w_ref, o_ref, oacc_ref):
    scale, shift = _scale_shift(acc_ref, gb_ref)
    zn = jnp.maximum(z_ref[...] * scale + shift, 0.0)
    o = jnp.dot(zn, w_ref[...], preferred_element_type=jnp.float32)
    o_ref[...] = o
    _stats_update(oacc_ref, o)


def _mm(z, acc, gb, wt, cin, cout):
    return pl.pallas_call(
        _mm_body,
        grid=(_GRID,),
        in_specs=[
            pl.BlockSpec((_RT, cin), lambda i: (i, 0)),
            pl.BlockSpec((2, cin), lambda i: (0, 0)),
            pl.BlockSpec((2, cin), lambda i: (0, 0)),
            pl.BlockSpec((cin, cout), lambda i: (0, 0)),
        ],
        out_specs=[
            pl.BlockSpec((_RT, cout), lambda i: (i, 0)),
            pl.BlockSpec((2, cout), lambda i: (0, 0)),
        ],
        out_shape=[jax.ShapeDtypeStruct((_ROWS, cout), jnp.float32),
                   jax.ShapeDtypeStruct((2, cout), jnp.float32)],
    )(z, acc, gb, wt)


def _m4_body(z_ref, acc_ref, gb_ref, o_ref):
    scale, shift = _scale_shift(acc_ref, gb_ref)
    zn = jnp.maximum(z_ref[...] * scale + shift, 0.0)
    o_ref[...] = jnp.max(zn.reshape(_RT // _K, _K, 256), axis=1)


def _m4(z, acc, gb):
    return pl.pallas_call(
        _m4_body,
        grid=(_GRID,),
        in_specs=[
            pl.BlockSpec((_RT, 256), lambda i: (i, 0)),
            pl.BlockSpec((2, 256), lambda i: (0, 0)),
            pl.BlockSpec((2, 256), lambda i: (0, 0)),
        ],
        out_specs=pl.BlockSpec((_RT // _K, 256), lambda i: (i, 0)),
        out_shape=jax.ShapeDtypeStruct((_B * _S, 256), jnp.float32),
    )(z, acc, gb)


def kernel(xyz, points, W0, b0, g0, beta0, W1, b1, g1, beta1,
           W2, b2, g2, beta2):
    xt = jnp.transpose(xyz, (2, 0, 1))
    far0 = jax.random.randint(jax.random.key(1), (_B,), 0, _N)
    far0 = far0.astype(jnp.int32)[:, None]
    nxt = _fps(xt, far0)

    pts = points.reshape(_B * _N, _CH)
    xg, xp = _scq(xt, nxt, pts)

    wg = jnp.zeros((16, 128), jnp.float32).at[0:3, :].set(W0[:, 0:3].T)
    wp = W0[:, 3:].T
    z0, acc0 = _m1(xg, xp, wg, wp)
    z1, acc1 = _mm(z0, acc0, jnp.stack([g0, beta0]), W1.T, 128, 128)
    z2, acc2 = _mm(z1, acc1, jnp.stack([g1, beta1]), W2.T, 128, 256)
    out = _m4(z2, acc2, jnp.stack([g2, beta2]))

    new_xyz = jnp.transpose(nxt, (1, 2, 0))
    return (new_xyz, out.reshape(_B, _S, 256))


# speedup vs baseline: 10.5791x; 1.0537x over previous
import functools

import jax
import jax.numpy as jnp
from jax import lax
from jax.experimental import pallas as pl
from jax.experimental.pallas import tpu as pltpu
from jax.experimental.pallas import tpu_sc as plsc

_B, _N, _S, _K = 8, 4096, 512, 32
_R2 = 0.2 * 0.2
_CH = 64
_NC, _NS = 2, 16
_NW = _NC * _NS
_CPW = (_B * _S) // _NW
_ROWS = _B * _S * _K
_RPW = _CPW * _K
_NCH = _N // 16


def _fps_body(xt_ref, f0_ref, nxt_ref, dist_ref):
    X = xt_ref[0]
    Y = xt_ref[1]
    Z = xt_ref[2]
    lane = lax.broadcasted_iota(jnp.int32, (_B, _N), 1)
    sidx = lax.broadcasted_iota(jnp.int32, (_B, _S), 1)
    dist_ref[...] = jnp.full((_B, _N), 1e10, jnp.float32)
    nxt_ref[...] = jnp.zeros((3, _B, _S), jnp.float32)

    def step(i, far):
        oh = lane == far
        cx = jnp.sum(jnp.where(oh, X, 0.0), axis=1, keepdims=True)
        cy = jnp.sum(jnp.where(oh, Y, 0.0), axis=1, keepdims=True)
        cz = jnp.sum(jnp.where(oh, Z, 0.0), axis=1, keepdims=True)
        rec = sidx == i
        nxt_ref[0] = jnp.where(rec, cx, nxt_ref[0])
        nxt_ref[1] = jnp.where(rec, cy, nxt_ref[1])
        nxt_ref[2] = jnp.where(rec, cz, nxt_ref[2])
        dx = X - cx
        dy = Y - cy
        dz = Z - cz
        d = dx * dx + dy * dy + dz * dz
        dm = jnp.minimum(dist_ref[...], d)
        dist_ref[...] = dm
        m = jnp.max(dm, axis=1, keepdims=True)
        return jnp.min(jnp.where(dm == m, lane, _N), axis=1, keepdims=True)

    lax.fori_loop(0, _S, step, f0_ref[...])


def _fps(xt, far0):
    return pl.pallas_call(
        _fps_body,
        out_shape=jax.ShapeDtypeStruct((3, _B, _S), jnp.float32),
        scratch_shapes=[pltpu.VMEM((_B, _N), jnp.float32)],
    )(xt, far0)


def _bf16r(x):
    b = plsc.bitcast(x, jnp.int32)
    r = (b + 0x7FFF + ((b >> 16) & 1)) & jnp.int32(-65536)
    return plsc.bitcast(r, jnp.float32)


def _scq_body(xt_hbm, nxt_hbm, pts_hbm, zrow_hbm, xg_hbm, xp_hbm,
              xs_v, ys_v, zs_v, x2_v, xb_v, yb_v, zb_v,
              cx_v, cy_v, cz_v,
              idxbuf_v, idx_all_v, xg_v, pbufa_v, pbufb_v, gsa, gsb):
    w = lax.axis_index("s") * _NC + lax.axis_index("c")
    b = w // (_NW // _B)
    s0 = (w % (_NW // _B)) * _CPW
    boff = b * _N

    pltpu.sync_copy(xt_hbm.at[0, b], xs_v)
    pltpu.sync_copy(xt_hbm.at[1, b], ys_v)
    pltpu.sync_copy(xt_hbm.at[2, b], zs_v)
    pltpu.sync_copy(nxt_hbm.at[0, b, pl.ds(s0, _CPW)], cx_v)
    pltpu.sync_copy(nxt_hbm.at[1, b, pl.ds(s0, _CPW)], cy_v)
    pltpu.sync_copy(nxt_hbm.at[2, b, pl.ds(s0, _CPW)], cz_v)
    pltpu.sync_copy(zrow_hbm, xg_v)

    def x2_step(t, _):
        sl = pl.ds(t * 16, 16)
        xs = xs_v[sl]
        ys = ys_v[sl]
        zs = zs_v[sl]
        x2_v[sl] = (xs * xs + ys * ys) + zs * zs
        xb_v[sl] = _bf16r(xs)
        yb_v[sl] = _bf16r(ys)
        zb_v[sl] = _bf16r(zs)
        return 0

    lax.fori_loop(0, _NCH, x2_step, 0)

    iota16 = lax.iota(jnp.int32, 16)

    def sel_step(cent, _):
        csplat = jnp.full((16,), cent, jnp.int32)
        cxs = plsc.load_gather(cx_v, [csplat])
        cys = plsc.load_gather(cy_v, [csplat])
        czs = plsc.load_gather(cz_v, [csplat])
        c2s = (cxs * cxs + cys * cys) + czs * czs
        cxb = _bf16r(cxs)
        cyb = _bf16r(cys)
        czb = _bf16r(czs)

        def cond(st):
            j, cnt = st
            return jnp.logical_and(cnt < _K, j < _NCH)

        def body(st):
            j, cnt = st
            sl = pl.ds(j * 16, 16)
            xb = xb_v[sl]
            yb = yb_v[sl]
            zb = zb_v[sl]
            x2 = x2_v[sl]
            dot = (cxb * xb + cyb * yb) + czb * zb
            sq = (c2s + x2) - 2.0 * dot
            mask = jnp.logical_not(sq > _R2)
            plsc.store_compressed(idxbuf_v.at[pl.ds(cnt, 16)],
                                  iota16 + j * 16, mask=mask)
            cnt = cnt + jnp.sum(jnp.where(mask, 1, 0))
            return j + 1, cnt

        _, cnt = lax.while_loop(cond, body, (0, 0))
        first = plsc.load_gather(idxbuf_v, [csplat * 0])
        row = cent // 4
        col = (cent % 4) * _K
        for h in range(2):
            v = idxbuf_v[pl.ds(h * 16, 16)]
            valid = (iota16 + h * 16) < cnt
            v = jnp.where(valid, v, first)
            gx = plsc.load_gather(xs_v, [v]) - cxs
            gy = plsc.load_gather(ys_v, [v]) - cys
            gz = plsc.load_gather(zs_v, [v]) - czs
            rows = iota16 + (cent * _K + h * 16)
            plsc.store_scatter(xg_v, [rows, jnp.zeros((16,), jnp.int32)], gx)
            plsc.store_scatter(xg_v, [rows, jnp.full((16,), 1, jnp.int32)], gy)
            plsc.store_scatter(xg_v, [rows, jnp.full((16,), 2, jnp.int32)], gz)
            idx_all_v[row, pl.ds(col + h * 16, 16)] = v + boff

        @pl.when(cent % 4 == 3)
        def _fire():
            r = cent // 4

            def drain_write(buf, gs, rr):
                pltpu.make_async_copy(
                    xp_hbm.at[pl.ds(0, 128)], buf, gs).wait()
                pltpu.sync_copy(
                    buf, xp_hbm.at[pl.ds(w * _RPW + rr * 128, 128)])

            @pl.when(r % 2 == 0)
            def _a():
                @pl.when(r >= 2)
                def _():
                    drain_write(pbufa_v, gsa, r - 2)
                pltpu.async_copy(pts_hbm.at[idx_all_v.at[r]], pbufa_v, gsa)

            @pl.when(r % 2 == 1)
            def _b():
                @pl.when(r >= 2)
                def _():
                    drain_write(pbufb_v, gsb, r - 2)
                pltpu.async_copy(pts_hbm.at[idx_all_v.at[r]], pbufb_v, gsb)

        return 0

    lax.fori_loop(0, _CPW, sel_step, 0)

    nrow = _RPW // 128
    pltpu.make_async_copy(xp_hbm.at[pl.ds(0, 128)], pbufa_v, gsa).wait()
    pltpu.sync_copy(pbufa_v, xp_hbm.at[pl.ds(w * _RPW + (nrow - 2) * 128, 128)])
    pltpu.make_async_copy(xp_hbm.at[pl.ds(0, 128)], pbufb_v, gsb).wait()
    pltpu.sync_copy(pbufb_v, xp_hbm.at[pl.ds(w * _RPW + (nrow - 1) * 128, 128)])
    pltpu.sync_copy(xg_v, xg_hbm.at[pl.ds(w * _RPW, _RPW)])


_scq_cache = []


def _scq(xt, nxt, pts):
    if not _scq_cache:
        _scq_cache.append(_build_scq())
    zrow = jnp.zeros((_RPW, 16), jnp.float32)
    return _scq_cache[0](xt, nxt, pts, zrow)


def _build_scq():
    return functools.partial(
        pl.kernel,
        out_type=[jax.ShapeDtypeStruct((_ROWS, 16), jnp.float32),
                  jax.ShapeDtypeStruct((_ROWS, _CH), jnp.float32)],
        mesh=plsc.VectorSubcoreMesh(core_axis_name="c", subcore_axis_name="s",
                                    num_cores=_NC, num_subcores=_NS),
        compiler_params=pltpu.CompilerParams(needs_layout_passes=False,
                                             use_tc_tiling_on_sc=False),
        scratch_types=[
        pltpu.VMEM((_N,), jnp.float32),
        pltpu.VMEM((_N,), jnp.float32),
        pltpu.VMEM((_N,), jnp.float32),
        pltpu.VMEM((_N,), jnp.float32),
        pltpu.VMEM((_N,), jnp.float32),
        pltpu.VMEM((_N,), jnp.float32),
        pltpu.VMEM((_N,), jnp.float32),
        pltpu.VMEM((_CPW,), jnp.float32),
        pltpu.VMEM((_CPW,), jnp.float32),
        pltpu.VMEM((_CPW,), jnp.float32),
        pltpu.VMEM((48,), jnp.int32),
        pltpu.VMEM((_RPW // 128, 128), jnp.int32),
        pltpu.VMEM((_RPW, 16), jnp.float32),
            pltpu.VMEM((128, _CH), jnp.float32),
            pltpu.VMEM((128, _CH), jnp.float32),
            pltpu.SemaphoreType.DMA,
            pltpu.SemaphoreType.DMA,
        ],
    )(_scq_body)


_RT = 1024
_GRID = _ROWS // _RT


def _stats_update(acc_ref, z):
    part = jnp.concatenate([jnp.sum(z, axis=0, keepdims=True),
                            jnp.sum(z * z, axis=0, keepdims=True)], axis=0)

    @pl.when(pl.program_id(0) == 0)
    def _():
        acc_ref[...] = jnp.zeros_like(acc_ref)

    acc_ref[...] += part


def _scale_shift(acc_ref, gb_ref):
    mean = acc_ref[0:1, :] * (1.0 / _ROWS)
    var = acc_ref[1:2, :] * (1.0 / _ROWS) - mean * mean
    scale = gb_ref[0:1, :] * lax.rsqrt(var + 1e-5)
    shift = gb_ref[1:2, :] - mean * scale
    return scale, shift


def _m1_body(xg_ref, xp_ref, wg_ref, wp_ref, z_ref, acc_ref):
    z = jnp.dot(xg_ref[...], wg_ref[...], preferred_element_type=jnp.float32)
    z = z + jnp.dot(xp_ref[...], wp_ref[...], preferred_element_type=jnp.float32)
    z_ref[...] = z
    _stats_update(acc_ref, z)


def _m1(xg, xp, wg, wp):
    return pl.pallas_call(
        _m1_body,
        grid=(_GRID,),
        in_specs=[
            pl.BlockSpec((_RT, 16), lambda i: (i, 0)),
            pl.BlockSpec((_RT, _CH), lambda i: (i, 0)),
            pl.BlockSpec((16, 128), lambda i: (0, 0)),
            pl.BlockSpec((_CH, 128), lambda i: (0, 0)),
        ],
        out_specs=[
            pl.BlockSpec((_RT, 128), lambda i: (i, 0)),
            pl.BlockSpec((2, 128), lambda i: (0, 0)),
        ],
        out_shape=[jax.ShapeDtypeStruct((_ROWS, 128), jnp.float32),
                   jax.ShapeDtypeStruct((2, 128), jnp.float32)],
    )(xg, xp, wg, wp)


def _mm_body(z_ref, acc_ref, gb_ref, w_ref, o_ref, oacc_ref):
    scale, shift = _scale_shift(acc_ref, gb_ref)
    zn = jnp.maximum(z_ref[...] * scale + shift, 0.0)
    o = jnp.dot(zn, w_ref[...], preferred_element_type=jnp.float32)
    o_ref[...] = o
    _stats_update(oacc_ref, o)


def _mm(z, acc, gb, wt, cin, cout):
    return pl.pallas_call(
        _mm_body,
        grid=(_GRID,),
        in_specs=[
            pl.BlockSpec((_RT, cin), lambda i: (i, 0)),
            pl.BlockSpec((2, cin), lambda i: (0, 0)),
            pl.BlockSpec((2, cin), lambda i: (0, 0)),
            pl.BlockSpec((cin, cout), lambda i: (0, 0)),
        ],
        out_specs=[
            pl.BlockSpec((_RT, cout), lambda i: (i, 0)),
            pl.BlockSpec((2, cout), lambda i: (0, 0)),
        ],
        out_shape=[jax.ShapeDtypeStruct((_ROWS, cout), jnp.float32),
                   jax.ShapeDtypeStruct((2, cout), jnp.float32)],
    )(z, acc, gb, wt)


def _m3s_body(z_ref, acc_ref, gb_ref, w_ref, oacc_ref):
    scale, shift = _scale_shift(acc_ref, gb_ref)
    zn = jnp.maximum(z_ref[...] * scale + shift, 0.0)
    o = jnp.dot(zn, w_ref[...], preferred_element_type=jnp.float32)
    _stats_update(oacc_ref, o)


def _m3s(z, acc, gb, wt):
    return pl.pallas_call(
        _m3s_body,
        grid=(_GRID,),
        in_specs=[
            pl.BlockSpec((_RT, 128), lambda i: (i, 0)),
            pl.BlockSpec((2, 128), lambda i: (0, 0)),
            pl.BlockSpec((2, 128), lambda i: (0, 0)),
            pl.BlockSpec((128, 256), lambda i: (0, 0)),
        ],
        out_specs=pl.BlockSpec((2, 256), lambda i: (0, 0)),
        out_shape=jax.ShapeDtypeStruct((2, 256), jnp.float32),
    )(z, acc, gb, wt)


def _m4_body(z_ref, acc_ref, gb_ref, w_ref, acc2_ref, gb2_ref, o_ref):
    scale, shift = _scale_shift(acc_ref, gb_ref)
    zn = jnp.maximum(z_ref[...] * scale + shift, 0.0)
    z2 = jnp.dot(zn, w_ref[...], preferred_element_type=jnp.float32)
    s2, sh2 = _scale_shift(acc2_ref, gb2_ref)
    z2n = jnp.maximum(z2 * s2 + sh2, 0.0)
    o_ref[...] = jnp.max(z2n.reshape(_RT // _K, _K, 256), axis=1)


def _m4(z, acc, gb, wt, acc2, gb2):
    return pl.pallas_call(
        _m4_body,
        grid=(_GRID,),
        in_specs=[
            pl.BlockSpec((_RT, 128), lambda i: (i, 0)),
            pl.BlockSpec((2, 128), lambda i: (0, 0)),
            pl.BlockSpec((2, 128), lambda i: (0, 0)),
            pl.BlockSpec((128, 256), lambda i: (0, 0)),
            pl.BlockSpec((2, 256), lambda i: (0, 0)),
            pl.BlockSpec((2, 256), lambda i: (0, 0)),
        ],
        out_specs=pl.BlockSpec((_RT // _K, 256), lambda i: (i, 0)),
        out_shape=jax.ShapeDtypeStruct((_B * _S, 256), jnp.float32),
    )(z, acc, gb, wt, acc2, gb2)


def kernel(xyz, points, W0, b0, g0, beta0, W1, b1, g1, beta1,
           W2, b2, g2, beta2):
    xt = jnp.transpose(xyz, (2, 0, 1))
    far0 = jax.random.randint(jax.random.key(1), (_B,), 0, _N)
    far0 = far0.astype(jnp.int32)[:, None]
    nxt = _fps(xt, far0)

    pts = points.reshape(_B * _N, _CH)
    xg, xp = _scq(xt, nxt, pts)

    wg = jnp.zeros((16, 128), jnp.float32).at[0:3, :].set(W0[:, 0:3].T)
    wp = W0[:, 3:].T
    z0, acc0 = _m1(xg, xp, wg, wp)
    gb0 = jnp.stack([g0, beta0])
    gb1 = jnp.stack([g1, beta1])
    gb2 = jnp.stack([g2, beta2])
    w2t = W2.T
    z1, acc1 = _mm(z0, acc0, gb0, W1.T, 128, 128)
    acc2 = _m3s(z1, acc1, gb1, w2t)
    out = _m4(z1, acc1, gb1, w2t, acc2, gb2)

    new_xyz = jnp.transpose(nxt, (1, 2, 0))
    return (new_xyz, out.reshape(_B, _S, 256))


# speedup vs baseline: 12.1066x; 1.1444x over previous
import functools

import jax
import jax.numpy as jnp
from jax import lax
from jax.experimental import pallas as pl
from jax.experimental.pallas import tpu as pltpu
from jax.experimental.pallas import tpu_sc as plsc

_B, _N, _S, _K = 8, 4096, 512, 32
_R2 = 0.2 * 0.2
_CH = 64
_NC, _NS = 2, 16
_NW = _NC * _NS
_CPW = (_B * _S) // _NW
_ROWS = _B * _S * _K
_RPW = _CPW * _K
_NCH = _N // 16


def _fps_body(xt_ref, f0_ref, nxt_ref, dist_ref):
    X = xt_ref[0]
    Y = xt_ref[1]
    Z = xt_ref[2]
    lane = lax.broadcasted_iota(jnp.int32, (_B, _N), 1)
    sidx = lax.broadcasted_iota(jnp.int32, (_B, _S), 1)
    dist_ref[...] = jnp.full((_B, _N), 1e10, jnp.float32)
    nxt_ref[...] = jnp.zeros((3, _B, _S), jnp.float32)

    def step(i, far):
        oh = lane == far
        cx = jnp.sum(jnp.where(oh, X, 0.0), axis=1, keepdims=True)
        cy = jnp.sum(jnp.where(oh, Y, 0.0), axis=1, keepdims=True)
        cz = jnp.sum(jnp.where(oh, Z, 0.0), axis=1, keepdims=True)
        rec = sidx == i
        nxt_ref[0] = jnp.where(rec, cx, nxt_ref[0])
        nxt_ref[1] = jnp.where(rec, cy, nxt_ref[1])
        nxt_ref[2] = jnp.where(rec, cz, nxt_ref[2])
        dx = X - cx
        dy = Y - cy
        dz = Z - cz
        d = dx * dx + dy * dy + dz * dz
        dm = jnp.minimum(dist_ref[...], d)
        dist_ref[...] = dm
        m = jnp.max(dm, axis=1, keepdims=True)
        return jnp.min(jnp.where(dm == m, lane, _N), axis=1, keepdims=True)

    lax.fori_loop(0, _S, step, f0_ref[...])


def _fps(xt, far0):
    return pl.pallas_call(
        _fps_body,
        out_shape=jax.ShapeDtypeStruct((3, _B, _S), jnp.float32),
        scratch_shapes=[pltpu.VMEM((_B, _N), jnp.float32)],
    )(xt, far0)


def _bf16r(x):
    b = plsc.bitcast(x, jnp.int32)
    r = (b + 0x7FFF + ((b >> 16) & 1)) & jnp.int32(-65536)
    return plsc.bitcast(r, jnp.float32)


def _scq_body(xt_hbm, nxt_hbm, pts_hbm, zrow_hbm, xg_hbm, xp_hbm,
              xs_v, ys_v, zs_v, x2_v, xb_v, yb_v, zb_v,
              cx_v, cy_v, cz_v,
              idxbuf_v, idx_all_v, xg_v, pbufa_v, pbufb_v, gsa, gsb):
    w = lax.axis_index("s") * _NC + lax.axis_index("c")
    b = w // (_NW // _B)
    s0 = (w % (_NW // _B)) * _CPW
    boff = b * _N

    pltpu.sync_copy(xt_hbm.at[0, b], xs_v)
    pltpu.sync_copy(xt_hbm.at[1, b], ys_v)
    pltpu.sync_copy(xt_hbm.at[2, b], zs_v)
    pltpu.sync_copy(nxt_hbm.at[0, b, pl.ds(s0, _CPW)], cx_v)
    pltpu.sync_copy(nxt_hbm.at[1, b, pl.ds(s0, _CPW)], cy_v)
    pltpu.sync_copy(nxt_hbm.at[2, b, pl.ds(s0, _CPW)], cz_v)
    pltpu.sync_copy(zrow_hbm, xg_v)

    def x2_step(t, _):
        sl = pl.ds(t * 16, 16)
        xs = xs_v[sl]
        ys = ys_v[sl]
        zs = zs_v[sl]
        x2_v[sl] = (xs * xs + ys * ys) + zs * zs
        xb_v[sl] = _bf16r(xs)
        yb_v[sl] = _bf16r(ys)
        zb_v[sl] = _bf16r(zs)
        return 0

    lax.fori_loop(0, _NCH, x2_step, 0)

    iota16 = lax.iota(jnp.int32, 16)

    def sel_step(cent, _):
        csplat = jnp.full((16,), cent, jnp.int32)
        cxs = plsc.load_gather(cx_v, [csplat])
        cys = plsc.load_gather(cy_v, [csplat])
        czs = plsc.load_gather(cz_v, [csplat])
        c2s = (cxs * cxs + cys * cys) + czs * czs
        cxb = _bf16r(cxs)
        cyb = _bf16r(cys)
        czb = _bf16r(czs)

        def cond(st):
            j, cnt = st
            return jnp.logical_and(cnt < _K, j < _NCH)

        def body(st):
            j, cnt = st
            for u in range(4):
                sl = pl.ds((j + u) * 16, 16)
                xb = xb_v[sl]
                yb = yb_v[sl]
                zb = zb_v[sl]
                x2 = x2_v[sl]
                dot = (cxb * xb + cyb * yb) + czb * zb
                sq = (c2s + x2) - 2.0 * dot
                mask = jnp.logical_not(sq > _R2)
                plsc.store_compressed(idxbuf_v.at[pl.ds(cnt, 16)],
                                      iota16 + (j + u) * 16, mask=mask)
                cnt = cnt + plsc.all_reduce_population_count(mask)[0]
            return j + 4, cnt

        _, cnt = lax.while_loop(cond, body, (0, 0))
        first = plsc.load_gather(idxbuf_v, [csplat * 0])
        row = cent // 4
        col = (cent % 4) * _K
        for h in range(2):
            v = idxbuf_v[pl.ds(h * 16, 16)]
            valid = (iota16 + h * 16) < cnt
            v = jnp.where(valid, v, first)
            gx = plsc.load_gather(xs_v, [v]) - cxs
            gy = plsc.load_gather(ys_v, [v]) - cys
            gz = plsc.load_gather(zs_v, [v]) - czs
            rows = iota16 + (cent * _K + h * 16)
            plsc.store_scatter(xg_v, [rows, jnp.zeros((16,), jnp.int32)], gx)
            plsc.store_scatter(xg_v, [rows, jnp.full((16,), 1, jnp.int32)], gy)
            plsc.store_scatter(xg_v, [rows, jnp.full((16,), 2, jnp.int32)], gz)
            idx_all_v[row, pl.ds(col + h * 16, 16)] = v + boff

        @pl.when(cent % 4 == 3)
        def _fire():
            r = cent // 4

            def drain_write(buf, gs, rr):
                pltpu.make_async_copy(
                    xp_hbm.at[pl.ds(0, 128)], buf, gs).wait()
                pltpu.sync_copy(
                    buf, xp_hbm.at[pl.ds(w * _RPW + rr * 128, 128)])

            @pl.when(r % 2 == 0)
            def _a():
                @pl.when(r >= 2)
                def _():
                    drain_write(pbufa_v, gsa, r - 2)
                pltpu.async_copy(pts_hbm.at[idx_all_v.at[r]], pbufa_v, gsa)

            @pl.when(r % 2 == 1)
            def _b():
                @pl.when(r >= 2)
                def _():
                    drain_write(pbufb_v, gsb, r - 2)
                pltpu.async_copy(pts_hbm.at[idx_all_v.at[r]], pbufb_v, gsb)

        return 0

    lax.fori_loop(0, _CPW, sel_step, 0)

    nrow = _RPW // 128
    pltpu.make_async_copy(xp_hbm.at[pl.ds(0, 128)], pbufa_v, gsa).wait()
    pltpu.sync_copy(pbufa_v, xp_hbm.at[pl.ds(w * _RPW + (nrow - 2) * 128, 128)])
    pltpu.make_async_copy(xp_hbm.at[pl.ds(0, 128)], pbufb_v, gsb).wait()
    pltpu.sync_copy(pbufb_v, xp_hbm.at[pl.ds(w * _RPW + (nrow - 1) * 128, 128)])
    pltpu.sync_copy(xg_v, xg_hbm.at[pl.ds(w * _RPW, _RPW)])


_scq_cache = []


def _scq(xt, nxt, pts):
    if not _scq_cache:
        _scq_cache.append(_build_scq())
    zrow = jnp.zeros((_RPW, 16), jnp.float32)
    return _scq_cache[0](xt, nxt, pts, zrow)


def _build_scq():
    return functools.partial(
        pl.kernel,
        out_type=[jax.ShapeDtypeStruct((_ROWS, 16), jnp.float32),
                  jax.ShapeDtypeStruct((_ROWS, _CH), jnp.float32)],
        mesh=plsc.VectorSubcoreMesh(core_axis_name="c", subcore_axis_name="s",
                                    num_cores=_NC, num_subcores=_NS),
        compiler_params=pltpu.CompilerParams(needs_layout_passes=False,
                                             use_tc_tiling_on_sc=False),
        scratch_types=[
        pltpu.VMEM((_N,), jnp.float32),
        pltpu.VMEM((_N,), jnp.float32),
        pltpu.VMEM((_N,), jnp.float32),
        pltpu.VMEM((_N,), jnp.float32),
        pltpu.VMEM((_N,), jnp.float32),
        pltpu.VMEM((_N,), jnp.float32),
        pltpu.VMEM((_N,), jnp.float32),
        pltpu.VMEM((_CPW,), jnp.float32),
        pltpu.VMEM((_CPW,), jnp.float32),
        pltpu.VMEM((_CPW,), jnp.float32),
        pltpu.VMEM((96,), jnp.int32),
        pltpu.VMEM((_RPW // 128, 128), jnp.int32),
        pltpu.VMEM((_RPW, 16), jnp.float32),
            pltpu.VMEM((128, _CH), jnp.float32),
            pltpu.VMEM((128, _CH), jnp.float32),
            pltpu.SemaphoreType.DMA,
            pltpu.SemaphoreType.DMA,
        ],
    )(_scq_body)


_RT = 1024
_GRID = _ROWS // _RT


def _stats_update(acc_ref, z):
    part = jnp.concatenate([jnp.sum(z, axis=0, keepdims=True),
                            jnp.sum(z * z, axis=0, keepdims=True)], axis=0)

    @pl.when(pl.program_id(0) == 0)
    def _():
        acc_ref[...] = jnp.zeros_like(acc_ref)

    acc_ref[...] += part


def _scale_shift(acc_ref, gb_ref):
    mean = acc_ref[0:1, :] * (1.0 / _ROWS)
    var = acc_ref[1:2, :] * (1.0 / _ROWS) - mean * mean
    scale = gb_ref[0:1, :] * lax.rsqrt(var + 1e-5)
    shift = gb_ref[1:2, :] - mean * scale
    return scale, shift


def _z0(xg_ref, xp_ref, wg_ref, wp_ref):
    z = jnp.dot(xg_ref[...], wg_ref[...], preferred_element_type=jnp.float32)
    return z + jnp.dot(xp_ref[...], wp_ref[...],
                       preferred_element_type=jnp.float32)


def _m1s_body(xg_ref, xp_ref, wg_ref, wp_ref, acc_ref):
    _stats_update(acc_ref, _z0(xg_ref, xp_ref, wg_ref, wp_ref))


def _m1s(xg, xp, wg, wp):
    return pl.pallas_call(
        _m1s_body,
        grid=(_GRID,),
        in_specs=[
            pl.BlockSpec((_RT, 16), lambda i: (i, 0)),
            pl.BlockSpec((_RT, _CH), lambda i: (i, 0)),
            pl.BlockSpec((16, 128), lambda i: (0, 0)),
            pl.BlockSpec((_CH, 128), lambda i: (0, 0)),
        ],
        out_specs=pl.BlockSpec((2, 128), lambda i: (0, 0)),
        out_shape=jax.ShapeDtypeStruct((2, 128), jnp.float32),
    )(xg, xp, wg, wp)


def _m2_body(xg_ref, xp_ref, wg_ref, wp_ref, acc_ref, gb_ref, w_ref,
             o_ref, oacc_ref):
    z = _z0(xg_ref, xp_ref, wg_ref, wp_ref)
    scale, shift = _scale_shift(acc_ref, gb_ref)
    zn = jnp.maximum(z * scale + shift, 0.0)
    o = jnp.dot(zn, w_ref[...], preferred_element_type=jnp.float32)
    o_ref[...] = o
    _stats_update(oacc_ref, o)


def _m2(xg, xp, wg, wp, acc, gb, wt):
    return pl.pallas_call(
        _m2_body,
        grid=(_GRID,),
        in_specs=[
            pl.BlockSpec((_RT, 16), lambda i: (i, 0)),
            pl.BlockSpec((_RT, _CH), lambda i: (i, 0)),
            pl.BlockSpec((16, 128), lambda i: (0, 0)),
            pl.BlockSpec((_CH, 128), lambda i: (0, 0)),
            pl.BlockSpec((2, 128), lambda i: (0, 0)),
            pl.BlockSpec((2, 128), lambda i: (0, 0)),
            pl.BlockSpec((128, 128), lambda i: (0, 0)),
        ],
        out_specs=[
            pl.BlockSpec((_RT, 128), lambda i: (i, 0)),
            pl.BlockSpec((2, 128), lambda i: (0, 0)),
        ],
        out_shape=[jax.ShapeDtypeStruct((_ROWS, 128), jnp.float32),
                   jax.ShapeDtypeStruct((2, 128), jnp.float32)],
    )(xg, xp, wg, wp, acc, gb, wt)


def _m3s_body(z_ref, acc_ref, gb_ref, w_ref, oacc_ref):
    scale, shift = _scale_shift(acc_ref, gb_ref)
    zn = jnp.maximum(z_ref[...] * scale + shift, 0.0)
    o = jnp.dot(zn, w_ref[...], preferred_element_type=jnp.float32)
    _stats_update(oacc_ref, o)


def _m3s(z, acc, gb, wt):
    return pl.pallas_call(
        _m3s_body,
        grid=(_GRID,),
        in_specs=[
            pl.BlockSpec((_RT, 128), lambda i: (i, 0)),
            pl.BlockSpec((2, 128), lambda i: (0, 0)),
            pl.BlockSpec((2, 128), lambda i: (0, 0)),
            pl.BlockSpec((128, 256), lambda i: (0, 0)),
        ],
        out_specs=pl.BlockSpec((2, 256), lambda i: (0, 0)),
        out_shape=jax.ShapeDtypeStruct((2, 256), jnp.float32),
    )(z, acc, gb, wt)


def _m4_body(z_ref, acc_ref, gb_ref, w_ref, acc2_ref, gb2_ref, o_ref):
    scale, shift = _scale_shift(acc_ref, gb_ref)
    zn = jnp.maximum(z_ref[...] * scale + shift, 0.0)
    z2 = jnp.dot(zn, w_ref[...], preferred_element_type=jnp.float32)
    s2, sh2 = _scale_shift(acc2_ref, gb2_ref)
    z2n = jnp.maximum(z2 * s2 + sh2, 0.0)
    o_ref[...] = jnp.max(z2n.reshape(_RT // _K, _K, 256), axis=1)


def _m4(z, acc, gb, wt, acc2, gb2):
    return pl.pallas_call(
        _m4_body,
        grid=(_GRID,),
        in_specs=[
            pl.BlockSpec((_RT, 128), lambda i: (i, 0)),
            pl.BlockSpec((2, 128), lambda i: (0, 0)),
            pl.BlockSpec((2, 128), lambda i: (0, 0)),
            pl.BlockSpec((128, 256), lambda i: (0, 0)),
            pl.BlockSpec((2, 256), lambda i: (0, 0)),
            pl.BlockSpec((2, 256), lambda i: (0, 0)),
        ],
        out_specs=pl.BlockSpec((_RT // _K, 256), lambda i: (i, 0)),
        out_shape=jax.ShapeDtypeStruct((_B * _S, 256), jnp.float32),
    )(z, acc, gb, wt, acc2, gb2)


def kernel(xyz, points, W0, b0, g0, beta0, W1, b1, g1, beta1,
           W2, b2, g2, beta2):
    xt = jnp.transpose(xyz, (2, 0, 1))
    far0 = jax.random.randint(jax.random.key(1), (_B,), 0, _N)
    far0 = far0.astype(jnp.int32)[:, None]
    nxt = _fps(xt, far0)

    pts = points.reshape(_B * _N, _CH)
    xg, xp = _scq(xt, nxt, pts)

    wg = jnp.zeros((16, 128), jnp.float32).at[0:3, :].set(W0[:, 0:3].T)
    wp = W0[:, 3:].T
    gb0 = jnp.stack([g0, beta0])
    gb1 = jnp.stack([g1, beta1])
    gb2 = jnp.stack([g2, beta2])
    w2t = W2.T
    acc0 = _m1s(xg, xp, wg, wp)
    z1, acc1 = _m2(xg, xp, wg, wp, acc0, gb0, W1.T)
    acc2 = _m3s(z1, acc1, gb1, w2t)
    out = _m4(z1, acc1, gb1, w2t, acc2, gb2)

    new_xyz = jnp.transpose(nxt, (1, 2, 0))
    return (new_xyz, out.reshape(_B, _S, 256))


# speedup vs baseline: 14.2193x; 1.1745x over previous
import functools

import jax
import jax.numpy as jnp
from jax import lax
from jax.experimental import pallas as pl
from jax.experimental.pallas import tpu as pltpu
from jax.experimental.pallas import tpu_sc as plsc

_B, _N, _S, _K = 8, 4096, 512, 32
_R2 = 0.2 * 0.2
_CH = 64
_NC, _NS = 2, 16
_NW = _NC * _NS
_CPW = (_B * _S) // _NW
_ROWS = _B * _S * _K
_RPW = _CPW * _K
_NCH = _N // 16


def _fps_body(xt_ref, f0_ref, nxt_ref, dist_ref):
    X = xt_ref[0]
    Y = xt_ref[1]
    Z = xt_ref[2]
    lane = lax.broadcasted_iota(jnp.int32, (_B, _N), 1)
    sidx = lax.broadcasted_iota(jnp.int32, (_B, _S), 1)
    dist_ref[...] = jnp.full((_B, _N), 1e10, jnp.float32)
    nxt_ref[...] = jnp.zeros((3, _B, _S), jnp.float32)

    def step(i, far):
        oh = lane == far
        c = jnp.sum(jnp.where(oh[None, :, :], xt_ref[...], 0.0),
                    axis=2, keepdims=True)
        cx = c[0]
        cy = c[1]
        cz = c[2]
        rec = sidx == i
        nxt_ref[0] = jnp.where(rec, cx, nxt_ref[0])
        nxt_ref[1] = jnp.where(rec, cy, nxt_ref[1])
        nxt_ref[2] = jnp.where(rec, cz, nxt_ref[2])
        dx = X - cx
        dy = Y - cy
        dz = Z - cz
        d = dx * dx + dy * dy + dz * dz
        dm = jnp.minimum(dist_ref[...], d)
        dist_ref[...] = dm
        m = jnp.max(dm, axis=1, keepdims=True)
        return jnp.min(jnp.where(dm == m, lane, _N), axis=1, keepdims=True)

    lax.fori_loop(0, _S, step, f0_ref[...])


def _fps(xt, far0):
    return pl.pallas_call(
        _fps_body,
        out_shape=jax.ShapeDtypeStruct((3, _B, _S), jnp.float32),
        scratch_shapes=[pltpu.VMEM((_B, _N), jnp.float32)],
    )(xt, far0)


def _bf16r(x):
    b = plsc.bitcast(x, jnp.int32)
    r = (b + 0x7FFF + ((b >> 16) & 1)) & jnp.int32(-65536)
    return plsc.bitcast(r, jnp.float32)


def _scq_body(xt_hbm, nxt_hbm, pts_hbm, zrow_hbm, xg_hbm, xp_hbm,
              xs_v, ys_v, zs_v, x2_v, xb_v, yb_v, zb_v,
              cx_v, cy_v, cz_v,
              idxbuf_v, idx_all_v, xg_v, pbufa_v, pbufb_v, gsa, gsb):
    w = lax.axis_index("s") * _NC + lax.axis_index("c")
    b = w // (_NW // _B)
    s0 = (w % (_NW // _B)) * _CPW
    boff = b * _N

    pltpu.sync_copy(xt_hbm.at[0, b], xs_v)
    pltpu.sync_copy(xt_hbm.at[1, b], ys_v)
    pltpu.sync_copy(xt_hbm.at[2, b], zs_v)
    pltpu.sync_copy(nxt_hbm.at[0, b, pl.ds(s0, _CPW)], cx_v)
    pltpu.sync_copy(nxt_hbm.at[1, b, pl.ds(s0, _CPW)], cy_v)
    pltpu.sync_copy(nxt_hbm.at[2, b, pl.ds(s0, _CPW)], cz_v)
    pltpu.sync_copy(zrow_hbm, xg_v)

    def x2_step(t, _):
        sl = pl.ds(t * 16, 16)
        xs = xs_v[sl]
        ys = ys_v[sl]
        zs = zs_v[sl]
        x2_v[sl] = (xs * xs + ys * ys) + zs * zs
        xb_v[sl] = _bf16r(xs)
        yb_v[sl] = _bf16r(ys)
        zb_v[sl] = _bf16r(zs)
        return 0

    lax.fori_loop(0, _NCH, x2_step, 0)

    iota16 = lax.iota(jnp.int32, 16)

    def sel_step(cent, _):
        csplat = jnp.full((16,), cent, jnp.int32)
        cxs = plsc.load_gather(cx_v, [csplat])
        cys = plsc.load_gather(cy_v, [csplat])
        czs = plsc.load_gather(cz_v, [csplat])
        c2s = (cxs * cxs + cys * cys) + czs * czs
        cxb = _bf16r(cxs)
        cyb = _bf16r(cys)
        czb = _bf16r(czs)

        def cond(st):
            j, cnt = st
            return jnp.logical_and(cnt < _K, j < _NCH)

        def body(st):
            j, cnt = st
            for u in range(8):
                sl = pl.ds((j + u) * 16, 16)
                xb = xb_v[sl]
                yb = yb_v[sl]
                zb = zb_v[sl]
                x2 = x2_v[sl]
                dot = (cxb * xb + cyb * yb) + czb * zb
                sq = (c2s + x2) - 2.0 * dot
                mask = jnp.logical_not(sq > _R2)
                plsc.store_compressed(idxbuf_v.at[pl.ds(cnt, 16)],
                                      iota16 + (j + u) * 16, mask=mask)
                cnt = cnt + plsc.all_reduce_population_count(mask)[0]
            return j + 8, cnt

        _, cnt = lax.while_loop(cond, body, (0, 0))
        first = plsc.load_gather(idxbuf_v, [csplat * 0])
        row = cent // 4
        col = (cent % 4) * _K
        for h in range(2):
            v = idxbuf_v[pl.ds(h * 16, 16)]
            valid = (iota16 + h * 16) < cnt
            v = jnp.where(valid, v, first)
            gx = plsc.load_gather(xs_v, [v]) - cxs
            gy = plsc.load_gather(ys_v, [v]) - cys
            gz = plsc.load_gather(zs_v, [v]) - czs
            rows = iota16 + (cent * _K + h * 16)
            plsc.store_scatter(xg_v, [rows, jnp.zeros((16,), jnp.int32)], gx)
            plsc.store_scatter(xg_v, [rows, jnp.full((16,), 1, jnp.int32)], gy)
            plsc.store_scatter(xg_v, [rows, jnp.full((16,), 2, jnp.int32)], gz)
            idx_all_v[row, pl.ds(col + h * 16, 16)] = v + boff

        @pl.when(cent % 4 == 3)
        def _fire():
            r = cent // 4

            def drain_write(buf, gs, rr):
                pltpu.make_async_copy(
                    xp_hbm.at[pl.ds(0, 128)], buf, gs).wait()
                pltpu.sync_copy(
                    buf, xp_hbm.at[pl.ds(w * _RPW + rr * 128, 128)])

            @pl.when(r % 2 == 0)
            def _a():
                @pl.when(r >= 2)
                def _():
                    drain_write(pbufa_v, gsa, r - 2)
                pltpu.async_copy(pts_hbm.at[idx_all_v.at[r]], pbufa_v, gsa)

            @pl.when(r % 2 == 1)
            def _b():
                @pl.when(r >= 2)
                def _():
                    drain_write(pbufb_v, gsb, r - 2)
                pltpu.async_copy(pts_hbm.at[idx_all_v.at[r]], pbufb_v, gsb)

        return 0

    lax.fori_loop(0, _CPW, sel_step, 0)

    nrow = _RPW // 128
    pltpu.make_async_copy(xp_hbm.at[pl.ds(0, 128)], pbufa_v, gsa).wait()
    pltpu.sync_copy(pbufa_v, xp_hbm.at[pl.ds(w * _RPW + (nrow - 2) * 128, 128)])
    pltpu.make_async_copy(xp_hbm.at[pl.ds(0, 128)], pbufb_v, gsb).wait()
    pltpu.sync_copy(pbufb_v, xp_hbm.at[pl.ds(w * _RPW + (nrow - 1) * 128, 128)])
    pltpu.sync_copy(xg_v, xg_hbm.at[pl.ds(w * _RPW, _RPW)])


_scq_cache = []


def _scq(xt, nxt, pts):
    if not _scq_cache:
        _scq_cache.append(_build_scq())
    zrow = jnp.zeros((_RPW, 16), jnp.float32)
    return _scq_cache[0](xt, nxt, pts, zrow)


def _build_scq():
    return functools.partial(
        pl.kernel,
        out_type=[jax.ShapeDtypeStruct((_ROWS, 16), jnp.float32),
                  jax.ShapeDtypeStruct((_ROWS, _CH), jnp.float32)],
        mesh=plsc.VectorSubcoreMesh(core_axis_name="c", subcore_axis_name="s",
                                    num_cores=_NC, num_subcores=_NS),
        compiler_params=pltpu.CompilerParams(needs_layout_passes=False,
                                             use_tc_tiling_on_sc=False),
        scratch_types=[
        pltpu.VMEM((_N,), jnp.float32),
        pltpu.VMEM((_N,), jnp.float32),
        pltpu.VMEM((_N,), jnp.float32),
        pltpu.VMEM((_N,), jnp.float32),
        pltpu.VMEM((_N,), jnp.float32),
        pltpu.VMEM((_N,), jnp.float32),
        pltpu.VMEM((_N,), jnp.float32),
        pltpu.VMEM((_CPW,), jnp.float32),
        pltpu.VMEM((_CPW,), jnp.float32),
        pltpu.VMEM((_CPW,), jnp.float32),
        pltpu.VMEM((160,), jnp.int32),
        pltpu.VMEM((_RPW // 128, 128), jnp.int32),
        pltpu.VMEM((_RPW, 16), jnp.float32),
            pltpu.VMEM((128, _CH), jnp.float32),
            pltpu.VMEM((128, _CH), jnp.float32),
            pltpu.SemaphoreType.DMA,
            pltpu.SemaphoreType.DMA,
        ],
    )(_scq_body)


_RT = 2048
_GRID = _ROWS // _RT


def _stats_update(acc_ref, z):
    part = jnp.concatenate([jnp.sum(z, axis=0, keepdims=True),
                            jnp.sum(z * z, axis=0, keepdims=True)], axis=0)

    @pl.when(pl.program_id(0) == 0)
    def _():
        acc_ref[...] = jnp.zeros_like(acc_ref)

    acc_ref[...] += part


def _scale_shift(acc_ref, gb_ref):
    mean = acc_ref[0:1, :] * (1.0 / _ROWS)
    var = acc_ref[1:2, :] * (1.0 / _ROWS) - mean * mean
    scale = gb_ref[0:1, :] * lax.rsqrt(var + 1e-5)
    shift = gb_ref[1:2, :] - mean * scale
    return scale, shift


def _z0(xg_ref, xp_ref, wg_ref, wp_ref):
    z = jnp.dot(xg_ref[...], wg_ref[...], preferred_element_type=jnp.float32)
    return z + jnp.dot(xp_ref[...], wp_ref[...],
                       preferred_element_type=jnp.float32)


def _m1s_body(xg_ref, xp_ref, wg_ref, wp_ref, acc_ref):
    _stats_update(acc_ref, _z0(xg_ref, xp_ref, wg_ref, wp_ref))


def _m1s(xg, xp, wg, wp):
    return pl.pallas_call(
        _m1s_body,
        grid=(_GRID,),
        in_specs=[
            pl.BlockSpec((_RT, 16), lambda i: (i, 0)),
            pl.BlockSpec((_RT, _CH), lambda i: (i, 0)),
            pl.BlockSpec((16, 128), lambda i: (0, 0)),
            pl.BlockSpec((_CH, 128), lambda i: (0, 0)),
        ],
        out_specs=pl.BlockSpec((2, 128), lambda i: (0, 0)),
        out_shape=jax.ShapeDtypeStruct((2, 128), jnp.float32),
    )(xg, xp, wg, wp)


def _m2_body(xg_ref, xp_ref, wg_ref, wp_ref, acc_ref, gb_ref, w_ref,
             o_ref, oacc_ref):
    z = _z0(xg_ref, xp_ref, wg_ref, wp_ref)
    scale, shift = _scale_shift(acc_ref, gb_ref)
    zn = jnp.maximum(z * scale + shift, 0.0)
    o = jnp.dot(zn, w_ref[...], preferred_element_type=jnp.float32)
    o_ref[...] = o
    _stats_update(oacc_ref, o)


def _m2(xg, xp, wg, wp, acc, gb, wt):
    return pl.pallas_call(
        _m2_body,
        grid=(_GRID,),
        in_specs=[
            pl.BlockSpec((_RT, 16), lambda i: (i, 0)),
            pl.BlockSpec((_RT, _CH), lambda i: (i, 0)),
            pl.BlockSpec((16, 128), lambda i: (0, 0)),
            pl.BlockSpec((_CH, 128), lambda i: (0, 0)),
            pl.BlockSpec((2, 128), lambda i: (0, 0)),
            pl.BlockSpec((2, 128), lambda i: (0, 0)),
            pl.BlockSpec((128, 128), lambda i: (0, 0)),
        ],
        out_specs=[
            pl.BlockSpec((_RT, 128), lambda i: (i, 0)),
            pl.BlockSpec((2, 128), lambda i: (0, 0)),
        ],
        out_shape=[jax.ShapeDtypeStruct((_ROWS, 128), jnp.float32),
                   jax.ShapeDtypeStruct((2, 128), jnp.float32)],
    )(xg, xp, wg, wp, acc, gb, wt)


def _m3s_body(z_ref, acc_ref, gb_ref, w_ref, oacc_ref):
    scale, shift = _scale_shift(acc_ref, gb_ref)
    zn = jnp.maximum(z_ref[...] * scale + shift, 0.0)
    o = jnp.dot(zn, w_ref[...], preferred_element_type=jnp.float32)
    _stats_update(oacc_ref, o)


def _m3s(z, acc, gb, wt):
    return pl.pallas_call(
        _m3s_body,
        grid=(_GRID,),
        in_specs=[
            pl.BlockSpec((_RT, 128), lambda i: (i, 0)),
            pl.BlockSpec((2, 128), lambda i: (0, 0)),
            pl.BlockSpec((2, 128), lambda i: (0, 0)),
            pl.BlockSpec((128, 256), lambda i: (0, 0)),
        ],
        out_specs=pl.BlockSpec((2, 256), lambda i: (0, 0)),
        out_shape=jax.ShapeDtypeStruct((2, 256), jnp.float32),
    )(z, acc, gb, wt)


def _m4_body(z_ref, acc_ref, gb_ref, w_ref, acc2_ref, gb2_ref, o_ref):
    scale, shift = _scale_shift(acc_ref, gb_ref)
    zn = jnp.maximum(z_ref[...] * scale + shift, 0.0)
    z2 = jnp.dot(zn, w_ref[...], preferred_element_type=jnp.float32)
    s2, sh2 = _scale_shift(acc2_ref, gb2_ref)
    z2n = jnp.maximum(z2 * s2 + sh2, 0.0)
    o_ref[...] = jnp.max(z2n.reshape(_RT // _K, _K, 256), axis=1)


def _m4(z, acc, gb, wt, acc2, gb2):
    return pl.pallas_call(
        _m4_body,
        grid=(_GRID,),
        in_specs=[
            pl.BlockSpec((_RT, 128), lambda i: (i, 0)),
            pl.BlockSpec((2, 128), lambda i: (0, 0)),
            pl.BlockSpec((2, 128), lambda i: (0, 0)),
            pl.BlockSpec((128, 256), lambda i: (0, 0)),
            pl.BlockSpec((2, 256), lambda i: (0, 0)),
            pl.BlockSpec((2, 256), lambda i: (0, 0)),
        ],
        out_specs=pl.BlockSpec((_RT // _K, 256), lambda i: (i, 0)),
        out_shape=jax.ShapeDtypeStruct((_B * _S, 256), jnp.float32),
    )(z, acc, gb, wt, acc2, gb2)


def kernel(xyz, points, W0, b0, g0, beta0, W1, b1, g1, beta1,
           W2, b2, g2, beta2):
    xt = jnp.transpose(xyz, (2, 0, 1))
    far0 = jax.random.randint(jax.random.key(1), (_B,), 0, _N)
    far0 = far0.astype(jnp.int32)[:, None]
    nxt = _fps(xt, far0)

    pts = points.reshape(_B * _N, _CH)
    xg, xp = _scq(xt, nxt, pts)

    wg = jnp.zeros((16, 128), jnp.float32).at[0:3, :].set(W0[:, 0:3].T)
    wp = W0[:, 3:].T
    gb0 = jnp.stack([g0, beta0])
    gb1 = jnp.stack([g1, beta1])
    gb2 = jnp.stack([g2, beta2])
    w2t = W2.T
    acc0 = _m1s(xg, xp, wg, wp)
    z1, acc1 = _m2(xg, xp, wg, wp, acc0, gb0, W1.T)
    acc2 = _m3s(z1, acc1, gb1, w2t)
    out = _m4(z1, acc1, gb1, w2t, acc2, gb2)

    new_xyz = jnp.transpose(nxt, (1, 2, 0))
    return (new_xyz, out.reshape(_B, _S, 256))


# speedup vs baseline: 15.5121x; 1.0909x over previous
import functools

import jax
import jax.numpy as jnp
from jax import lax
from jax.experimental import pallas as pl
from jax.experimental.pallas import tpu as pltpu
from jax.experimental.pallas import tpu_sc as plsc

_B, _N, _S, _K = 8, 4096, 512, 32
_R2 = 0.2 * 0.2
_CH = 64
_NC, _NS = 2, 16
_NW = _NC * _NS
_CPW = (_B * _S) // _NW
_ROWS = _B * _S * _K
_RPW = _CPW * _K
_NCH = _N // 16


def _fps_body(xt_ref, f0_ref, nxt_ref, dist_ref):
    X = xt_ref[0]
    Y = xt_ref[1]
    Z = xt_ref[2]
    lane = lax.broadcasted_iota(jnp.int32, (_B, _N), 1)
    dist_ref[...] = jnp.full((_B, _N), 1e10, jnp.float32)

    def step(i, far):
        oh = lane == far
        c = jnp.sum(jnp.where(oh[None, :, :], xt_ref[...], 0.0),
                    axis=2, keepdims=True)
        nxt_ref[pl.ds(i, 1)] = c.reshape(1, 3, _B)
        cx = c[0]
        cy = c[1]
        cz = c[2]
        dx = X - cx
        dy = Y - cy
        dz = Z - cz
        d = dx * dx + dy * dy + dz * dz
        dm = jnp.minimum(dist_ref[...], d)
        dist_ref[...] = dm
        m = jnp.max(dm, axis=1, keepdims=True)
        return jnp.min(jnp.where(dm == m, lane, _N), axis=1, keepdims=True)

    lax.fori_loop(0, _S, step, f0_ref[...])


def _fps(xt, far0):
    return pl.pallas_call(
        _fps_body,
        out_shape=jax.ShapeDtypeStruct((_S, 3, _B), jnp.float32),
        scratch_shapes=[pltpu.VMEM((_B, _N), jnp.float32)],
    )(xt, far0)


def _bf16r(x):
    b = plsc.bitcast(x, jnp.int32)
    r = (b + 0x7FFF + ((b >> 16) & 1)) & jnp.int32(-65536)
    return plsc.bitcast(r, jnp.float32)


def _scq_body(xt_hbm, nxt_hbm, pts_hbm, zrow_hbm, xg_hbm, xp_hbm,
              xs_v, ys_v, zs_v, x2_v, xb_v, yb_v, zb_v,
              cx_v, cy_v, cz_v,
              idxbuf_v, idx_all_v, xg_v, pbufa_v, pbufb_v, gsa, gsb):
    w = lax.axis_index("s") * _NC + lax.axis_index("c")
    b = w // (_NW // _B)
    s0 = (w % (_NW // _B)) * _CPW
    boff = b * _N

    pltpu.sync_copy(xt_hbm.at[0, b], xs_v)
    pltpu.sync_copy(xt_hbm.at[1, b], ys_v)
    pltpu.sync_copy(xt_hbm.at[2, b], zs_v)
    pltpu.sync_copy(nxt_hbm.at[0, b, pl.ds(s0, _CPW)], cx_v)
    pltpu.sync_copy(nxt_hbm.at[1, b, pl.ds(s0, _CPW)], cy_v)
    pltpu.sync_copy(nxt_hbm.at[2, b, pl.ds(s0, _CPW)], cz_v)
    pltpu.sync_copy(zrow_hbm, xg_v)

    def x2_step(t, _):
        sl = pl.ds(t * 16, 16)
        xs = xs_v[sl]
        ys = ys_v[sl]
        zs = zs_v[sl]
        x2_v[sl] = (xs * xs + ys * ys) + zs * zs
        xb_v[sl] = _bf16r(xs)
        yb_v[sl] = _bf16r(ys)
        zb_v[sl] = _bf16r(zs)
        return 0

    lax.fori_loop(0, _NCH, x2_step, 0)

    iota16 = lax.iota(jnp.int32, 16)

    def sel_step(cent, _):
        csplat = jnp.full((16,), cent, jnp.int32)
        cxs = plsc.load_gather(cx_v, [csplat])
        cys = plsc.load_gather(cy_v, [csplat])
        czs = plsc.load_gather(cz_v, [csplat])
        c2s = (cxs * cxs + cys * cys) + czs * czs
        cxb = _bf16r(cxs)
        cyb = _bf16r(cys)
        czb = _bf16r(czs)

        def cond(st):
            j, cnt = st
            return jnp.logical_and(cnt < _K, j < _NCH)

        def body(st):
            j, cnt = st
            for u in range(8):
                sl = pl.ds((j + u) * 16, 16)
                xb = xb_v[sl]
                yb = yb_v[sl]
                zb = zb_v[sl]
                x2 = x2_v[sl]
                dot = (cxb * xb + cyb * yb) + czb * zb
                sq = (c2s + x2) - 2.0 * dot
                mask = jnp.logical_not(sq > _R2)
                plsc.store_compressed(idxbuf_v.at[pl.ds(cnt, 16)],
                                      iota16 + (j + u) * 16, mask=mask)
                cnt = cnt + plsc.all_reduce_population_count(mask)[0]
            return j + 8, cnt

        _, cnt = lax.while_loop(cond, body, (0, 0))
        first = plsc.load_gather(idxbuf_v, [csplat * 0])
        row = cent // 4
        col = (cent % 4) * _K
        for h in range(2):
            v = idxbuf_v[pl.ds(h * 16, 16)]
            valid = (iota16 + h * 16) < cnt
            v = jnp.where(valid, v, first)
            gx = plsc.load_gather(xs_v, [v]) - cxs
            gy = plsc.load_gather(ys_v, [v]) - cys
            gz = plsc.load_gather(zs_v, [v]) - czs
            rows = iota16 + (cent * _K + h * 16)
            plsc.store_scatter(xg_v, [rows, jnp.zeros((16,), jnp.int32)], gx)
            plsc.store_scatter(xg_v, [rows, jnp.full((16,), 1, jnp.int32)], gy)
            plsc.store_scatter(xg_v, [rows, jnp.full((16,), 2, jnp.int32)], gz)
            idx_all_v[row, pl.ds(col + h * 16, 16)] = v + boff

        @pl.when(cent % 4 == 3)
        def _fire():
            r = cent // 4

            def drain_write(buf, gs, rr):
                pltpu.make_async_copy(
                    xp_hbm.at[pl.ds(0, 128)], buf, gs).wait()
                pltpu.sync_copy(
                    buf, xp_hbm.at[pl.ds(w * _RPW + rr * 128, 128)])

            @pl.when(r % 2 == 0)
            def _a():
                @pl.when(r >= 2)
                def _():
                    drain_write(pbufa_v, gsa, r - 2)
                pltpu.async_copy(pts_hbm.at[idx_all_v.at[r]], pbufa_v, gsa)

            @pl.when(r % 2 == 1)
            def _b():
                @pl.when(r >= 2)
                def _():
                    drain_write(pbufb_v, gsb, r - 2)
                pltpu.async_copy(pts_hbm.at[idx_all_v.at[r]], pbufb_v, gsb)

        return 0

    lax.fori_loop(0, _CPW, sel_step, 0)

    nrow = _RPW // 128
    pltpu.make_async_copy(xp_hbm.at[pl.ds(0, 128)], pbufa_v, gsa).wait()
    pltpu.sync_copy(pbufa_v, xp_hbm.at[pl.ds(w * _RPW + (nrow - 2) * 128, 128)])
    pltpu.make_async_copy(xp_hbm.at[pl.ds(0, 128)], pbufb_v, gsb).wait()
    pltpu.sync_copy(pbufb_v, xp_hbm.at[pl.ds(w * _RPW + (nrow - 1) * 128, 128)])
    pltpu.sync_copy(xg_v, xg_hbm.at[pl.ds(w * _RPW, _RPW)])


_scq_cache = []


def _scq(xt, nxt, pts):
    if not _scq_cache:
        _scq_cache.append(_build_scq())
    zrow = jnp.zeros((_RPW, 8), jnp.float32)
    return _scq_cache[0](xt, nxt, pts, zrow)


def _build_scq():
    return functools.partial(
        pl.kernel,
        out_type=[jax.ShapeDtypeStruct((_ROWS, 8), jnp.float32),
                  jax.ShapeDtypeStruct((_ROWS, _CH), jnp.float32)],
        mesh=plsc.VectorSubcoreMesh(core_axis_name="c", subcore_axis_name="s",
                                    num_cores=_NC, num_subcores=_NS),
        compiler_params=pltpu.CompilerParams(needs_layout_passes=False,
                                             use_tc_tiling_on_sc=False),
        scratch_types=[
        pltpu.VMEM((_N,), jnp.float32),
        pltpu.VMEM((_N,), jnp.float32),
        pltpu.VMEM((_N,), jnp.float32),
        pltpu.VMEM((_N,), jnp.float32),
        pltpu.VMEM((_N,), jnp.float32),
        pltpu.VMEM((_N,), jnp.float32),
        pltpu.VMEM((_N,), jnp.float32),
        pltpu.VMEM((_CPW,), jnp.float32),
        pltpu.VMEM((_CPW,), jnp.float32),
        pltpu.VMEM((_CPW,), jnp.float32),
        pltpu.VMEM((160,), jnp.int32),
        pltpu.VMEM((_RPW // 128, 128), jnp.int32),
        pltpu.VMEM((_RPW, 8), jnp.float32),
            pltpu.VMEM((128, _CH), jnp.float32),
            pltpu.VMEM((128, _CH), jnp.float32),
            pltpu.SemaphoreType.DMA,
            pltpu.SemaphoreType.DMA,
        ],
    )(_scq_body)


_RT = 4096
_GRID = _ROWS // _RT


def _stats_update(acc_ref, z):
    part = jnp.concatenate([jnp.sum(z, axis=0, keepdims=True),
                            jnp.sum(z * z, axis=0, keepdims=True)], axis=0)

    @pl.when(pl.program_id(0) == 0)
    def _():
        acc_ref[...] = jnp.zeros_like(acc_ref)

    acc_ref[...] += part


def _scale_shift(acc_ref, gb_ref):
    mean = acc_ref[0:1, :] * (1.0 / _ROWS)
    var = acc_ref[1:2, :] * (1.0 / _ROWS) - mean * mean
    scale = gb_ref[0:1, :] * lax.rsqrt(var + 1e-5)
    shift = gb_ref[1:2, :] - mean * scale
    return scale, shift


def _z0(xg_ref, xp_ref, wg_ref, wp_ref):
    z = jnp.dot(xg_ref[...], wg_ref[...], preferred_element_type=jnp.float32)
    return z + jnp.dot(xp_ref[...], wp_ref[...],
                       preferred_element_type=jnp.float32)


def _m1s_body(xg_ref, xp_ref, wg_ref, wp_ref, acc_ref):
    _stats_update(acc_ref, _z0(xg_ref, xp_ref, wg_ref, wp_ref))


def _m1s(xg, xp, wg, wp):
    return pl.pallas_call(
        _m1s_body,
        grid=(_GRID,),
        in_specs=[
            pl.BlockSpec((_RT, 8), lambda i: (i, 0)),
            pl.BlockSpec((_RT, _CH), lambda i: (i, 0)),
            pl.BlockSpec((8, 128), lambda i: (0, 0)),
            pl.BlockSpec((_CH, 128), lambda i: (0, 0)),
        ],
        out_specs=pl.BlockSpec((2, 128), lambda i: (0, 0)),
        out_shape=jax.ShapeDtypeStruct((2, 128), jnp.float32),
    )(xg, xp, wg, wp)


def _m2_body(xg_ref, xp_ref, wg_ref, wp_ref, acc_ref, gb_ref, w_ref,
             o_ref, oacc_ref):
    z = _z0(xg_ref, xp_ref, wg_ref, wp_ref)
    scale, shift = _scale_shift(acc_ref, gb_ref)
    zn = jnp.maximum(z * scale + shift, 0.0)
    o = jnp.dot(zn, w_ref[...], preferred_element_type=jnp.float32)
    o_ref[...] = o
    _stats_update(oacc_ref, o)


def _m2(xg, xp, wg, wp, acc, gb, wt):
    return pl.pallas_call(
        _m2_body,
        grid=(_GRID,),
        in_specs=[
            pl.BlockSpec((_RT, 8), lambda i: (i, 0)),
            pl.BlockSpec((_RT, _CH), lambda i: (i, 0)),
            pl.BlockSpec((8, 128), lambda i: (0, 0)),
            pl.BlockSpec((_CH, 128), lambda i: (0, 0)),
            pl.BlockSpec((2, 128), lambda i: (0, 0)),
            pl.BlockSpec((2, 128), lambda i: (0, 0)),
            pl.BlockSpec((128, 128), lambda i: (0, 0)),
        ],
        out_specs=[
            pl.BlockSpec((_RT, 128), lambda i: (i, 0)),
            pl.BlockSpec((2, 128), lambda i: (0, 0)),
        ],
        out_shape=[jax.ShapeDtypeStruct((_ROWS, 128), jnp.float32),
                   jax.ShapeDtypeStruct((2, 128), jnp.float32)],
    )(xg, xp, wg, wp, acc, gb, wt)


def _m3s_body(z_ref, acc_ref, gb_ref, w_ref, oacc_ref):
    scale, shift = _scale_shift(acc_ref, gb_ref)
    zn = jnp.maximum(z_ref[...] * scale + shift, 0.0)
    o = jnp.dot(zn, w_ref[...], preferred_element_type=jnp.float32)
    _stats_update(oacc_ref, o)


def _m3s(z, acc, gb, wt):
    return pl.pallas_call(
        _m3s_body,
        grid=(_GRID,),
        in_specs=[
            pl.BlockSpec((_RT, 128), lambda i: (i, 0)),
            pl.BlockSpec((2, 128), lambda i: (0, 0)),
            pl.BlockSpec((2, 128), lambda i: (0, 0)),
            pl.BlockSpec((128, 256), lambda i: (0, 0)),
        ],
        out_specs=pl.BlockSpec((2, 256), lambda i: (0, 0)),
        out_shape=jax.ShapeDtypeStruct((2, 256), jnp.float32),
    )(z, acc, gb, wt)


def _m4_body(z_ref, acc_ref, gb_ref, w_ref, acc2_ref, gb2_ref, o_ref):
    scale, shift = _scale_shift(acc_ref, gb_ref)
    zn = jnp.maximum(z_ref[...] * scale + shift, 0.0)
    z2 = jnp.dot(zn, w_ref[...], preferred_element_type=jnp.float32)
    s2, sh2 = _scale_shift(acc2_ref, gb2_ref)
    z2n = jnp.maximum(z2 * s2 + sh2, 0.0)
    o_ref[...] = jnp.max(z2n.reshape(_RT // _K, _K, 256), axis=1)


def _m4(z, acc, gb, wt, acc2, gb2):
    return pl.pallas_call(
        _m4_body,
        grid=(_GRID,),
        in_specs=[
            pl.BlockSpec((_RT, 128), lambda i: (i, 0)),
            pl.BlockSpec((2, 128), lambda i: (0, 0)),
            pl.BlockSpec((2, 128), lambda i: (0, 0)),
            pl.BlockSpec((128, 256), lambda i: (0, 0)),
            pl.BlockSpec((2, 256), lambda i: (0, 0)),
            pl.BlockSpec((2, 256), lambda i: (0, 0)),
        ],
        out_specs=pl.BlockSpec((_RT // _K, 256), lambda i: (i, 0)),
        out_shape=jax.ShapeDtypeStruct((_B * _S, 256), jnp.float32),
    )(z, acc, gb, wt, acc2, gb2)


def kernel(xyz, points, W0, b0, g0, beta0, W1, b1, g1, beta1,
           W2, b2, g2, beta2):
    xt = jnp.transpose(xyz, (2, 0, 1))
    far0 = jax.random.randint(jax.random.key(1), (_B,), 0, _N)
    far0 = far0.astype(jnp.int32)[:, None]
    nxt2 = _fps(xt, far0)
    nxt = jnp.transpose(nxt2, (1, 2, 0))

    pts = points.reshape(_B * _N, _CH)
    xg, xp = _scq(xt, nxt, pts)

    wg = jnp.zeros((8, 128), jnp.float32).at[0:3, :].set(W0[:, 0:3].T)
    wp = W0[:, 3:].T
    gb0 = jnp.stack([g0, beta0])
    gb1 = jnp.stack([g1, beta1])
    gb2 = jnp.stack([g2, beta2])
    w2t = W2.T
    acc0 = _m1s(xg, xp, wg, wp)
    z1, acc1 = _m2(xg, xp, wg, wp, acc0, gb0, W1.T)
    acc2 = _m3s(z1, acc1, gb1, w2t)
    out = _m4(z1, acc1, gb1, w2t, acc2, gb2)

    new_xyz = jnp.transpose(nxt2, (2, 0, 1))
    return (new_xyz, out.reshape(_B, _S, 256))
